# baseline (speedup 1.0000x reference)
import functools

import jax
import jax.numpy as jnp
from jax import lax
from jax.experimental import pallas as pl
from jax.experimental.pallas import tpu as pltpu
from jax.experimental.pallas import tpu_sc as plsc

_VOCAB = 1000000
_EMBED_DIM = 64
_BATCH = 16384
_HIST = 50

_NC = 2
_NS = 16
_NW = _NC * _NS
_BB = 128
_NBLK = _BATCH // _BB
_KPW = _NBLK // _NW
_ITERS = _KPW * _HIST
_NTILE = _HIST * (_EMBED_DIM // 8) * _NBLK


def _transpose_block(gbuf, tbuf, idx_v, i):
    iota = lax.iota(jnp.int32, 16)
    for j in range(8):
        rows = iota + 16 * j
        par = lax.shift_left(lax.bitwise_and(idx_v[i, pl.ds(16 * j, 16)], 1), 6)
        for e in range(_EMBED_DIM):
            v = plsc.load_gather(gbuf, [rows, par + e])
            tbuf[e, pl.ds(16 * j, 16)] = v


def _embed_kernel(idx_hbm, w2_hbm, l_hbm,
                  idx_v, qb0, qb1, gb0, gb1, tb0, tb1,
                  gsem0, gsem1, wsem0, wsem1):
    wid = lax.axis_index("s") * _NC + lax.axis_index("c")
    qbs, gbs, tbs = (qb0, qb1), (gb0, gb1), (tb0, tb1)
    gsems, wsems = (gsem0, gsem1), (wsem0, wsem1)

    pltpu.sync_copy(idx_hbm.at[wid], idx_v)

    def fire_gather(i, p):
        for j in range(8):
            t = idx_v[i, pl.ds(16 * j, 16)]
            qbs[p][pl.ds(16 * j, 16)] = lax.shift_right_logical(t, 1)
        pltpu.async_copy(w2_hbm.at[qbs[p]], gbs[p], gsems[p])

    fire_gather(0, 0)
    fire_gather(1, 1)

    def step(i2, carry):
        for p in range(2):
            i = 2 * i2 + p
            k = i // _HIST
            h = i - k * _HIST
            bb = _KPW * wid + k
            pltpu.make_async_copy(
                w2_hbm.at[pl.ds(0, _BB)], gbs[p], gsems[p]
            ).wait()

            @pl.when(i2 >= 1)
            def _():
                for _eb in range(8):
                    pltpu.make_async_copy(
                        tbs[p].at[pl.ds(8 * _eb, 8)], l_hbm.at[0], wsems[p]
                    ).wait()

            _transpose_block(gbs[p], tbs[p], idx_v, i)

            for eb in range(8):
                pltpu.async_copy(
                    tbs[p].at[pl.ds(8 * eb, 8)],
                    l_hbm.at[h * 1024 + eb * 128 + bb],
                    wsems[p],
                )

            @pl.when(i + 2 < _ITERS)
            def _():
                fire_gather(i + 2, p)

        return carry

    lax.fori_loop(0, _ITERS // 2, step, 0)

    for p in range(2):
        for _eb in range(8):
            pltpu.make_async_copy(
                tbs[p].at[pl.ds(8 * _eb, 8)], l_hbm.at[0], wsems[p]
            ).wait()


@jax.jit
def _embed(idxP, w2):
    mesh = plsc.VectorSubcoreMesh(core_axis_name="c", subcore_axis_name="s")
    f = functools.partial(
        pl.kernel,
        mesh=mesh,
        out_type=jax.ShapeDtypeStruct((_NTILE, 8, 128), jnp.float32),
        scratch_types=[
            pltpu.VMEM((_ITERS, _BB), jnp.int32),
            pltpu.VMEM((_BB,), jnp.int32),
            pltpu.VMEM((_BB,), jnp.int32),
            pltpu.VMEM((_BB, 128), jnp.float32),
            pltpu.VMEM((_BB, 128), jnp.float32),
            pltpu.VMEM((_EMBED_DIM, _BB), jnp.float32),
            pltpu.VMEM((_EMBED_DIM, _BB), jnp.float32),
            pltpu.SemaphoreType.DMA,
            pltpu.SemaphoreType.DMA,
            pltpu.SemaphoreType.DMA,
            pltpu.SemaphoreType.DMA,
        ],
        compiler_params=pltpu.CompilerParams(needs_layout_passes=False),
    )(_embed_kernel)
    return f(idxP, w2)


def kernel(tokenIndex, weights):
    idx = tokenIndex.astype(jnp.int32)
    idxP = (idx.T.reshape(_HIST, _NBLK, _BB).transpose(1, 0, 2)
            .reshape(_NW, _ITERS, _BB))
    w2 = weights.reshape(_VOCAB // 2, 128)
    L = _embed(idxP, w2)
    return (
        L.reshape(_HIST, 8, _NBLK, 8, _BB)
        .transpose(2, 4, 0, 1, 3)
        .reshape(_BATCH, _HIST, _EMBED_DIM)
    )


# speedup vs baseline: 1.5474x; 1.5474x over previous
import functools

import jax
import jax.numpy as jnp
from jax import lax
from jax.experimental import pallas as pl
from jax.experimental.pallas import tpu as pltpu
from jax.experimental.pallas import tpu_sc as plsc

_VOCAB = 1000000
_EMBED_DIM = 64
_BATCH = 16384
_HIST = 50

_NC = 2
_NS = 16
_NW = _NC * _NS
_BB = 128
_NBLK = _BATCH // _BB
_KPW = _NBLK // _NW
_ITERS = _KPW * _HIST
_NTILE = _HIST * (_EMBED_DIM // 8) * _NBLK


def _transpose_block(gbuf, tbuf, idx_v, i):
    iota = lax.iota(jnp.int32, 16)
    rows = [iota + 16 * j for j in range(8)]
    pars = [
        lax.shift_left(lax.bitwise_and(idx_v[i, pl.ds(16 * j, 16)], 1), 6)
        for j in range(8)
    ]

    @plsc.parallel_loop(0, _EMBED_DIM, unroll=4)
    def _(e):
        for j in range(8):
            v = plsc.load_gather(gbuf, [rows[j], pars[j] + e])
            tbuf[e, pl.ds(16 * j, 16)] = v


def _embed_kernel(idx_hbm, w2_hbm, l_hbm,
                  idx_v, qb0, qb1, gb0, gb1, tb0, tb1,
                  gsem0, gsem1, wsem0, wsem1):
    wid = lax.axis_index("s") * _NC + lax.axis_index("c")
    qbs, gbs, tbs = (qb0, qb1), (gb0, gb1), (tb0, tb1)
    gsems, wsems = (gsem0, gsem1), (wsem0, wsem1)

    pltpu.sync_copy(idx_hbm.at[wid], idx_v)

    def fire_gather(i, p):
        for j in range(8):
            t = idx_v[i, pl.ds(16 * j, 16)]
            qbs[p][pl.ds(16 * j, 16)] = lax.shift_right_logical(t, 1)
        pltpu.async_copy(w2_hbm.at[qbs[p]], gbs[p], gsems[p])

    fire_gather(0, 0)
    fire_gather(1, 1)

    def step(i2, carry):
        for p in range(2):
            i = 2 * i2 + p
            k = i // _HIST
            h = i - k * _HIST
            bb = _KPW * wid + k
            pltpu.make_async_copy(
                w2_hbm.at[pl.ds(0, _BB)], gbs[p], gsems[p]
            ).wait()

            @pl.when(i2 >= 1)
            def _():
                for _eb in range(8):
                    pltpu.make_async_copy(
                        tbs[p].at[pl.ds(8 * _eb, 8)], l_hbm.at[0], wsems[p]
                    ).wait()

            _transpose_block(gbs[p], tbs[p], idx_v, i)

            for eb in range(8):
                pltpu.async_copy(
                    tbs[p].at[pl.ds(8 * eb, 8)],
                    l_hbm.at[h * 1024 + eb * 128 + bb],
                    wsems[p],
                )

            @pl.when(i + 2 < _ITERS)
            def _():
                fire_gather(i + 2, p)

        return carry

    lax.fori_loop(0, _ITERS // 2, step, 0)

    for p in range(2):
        for _eb in range(8):
            pltpu.make_async_copy(
                tbs[p].at[pl.ds(8 * _eb, 8)], l_hbm.at[0], wsems[p]
            ).wait()


@jax.jit
def _embed(idxP, w2):
    mesh = plsc.VectorSubcoreMesh(core_axis_name="c", subcore_axis_name="s")
    f = functools.partial(
        pl.kernel,
        mesh=mesh,
        out_type=jax.ShapeDtypeStruct((_NTILE, 8, 128), jnp.float32),
        scratch_types=[
            pltpu.VMEM((_ITERS, _BB), jnp.int32),
            pltpu.VMEM((_BB,), jnp.int32),
            pltpu.VMEM((_BB,), jnp.int32),
            pltpu.VMEM((_BB, 128), jnp.float32),
            pltpu.VMEM((_BB, 128), jnp.float32),
            pltpu.VMEM((_EMBED_DIM, _BB), jnp.float32),
            pltpu.VMEM((_EMBED_DIM, _BB), jnp.float32),
            pltpu.SemaphoreType.DMA,
            pltpu.SemaphoreType.DMA,
            pltpu.SemaphoreType.DMA,
            pltpu.SemaphoreType.DMA,
        ],
        compiler_params=pltpu.CompilerParams(needs_layout_passes=False),
    )(_embed_kernel)
    return f(idxP, w2)


def kernel(tokenIndex, weights):
    idx = tokenIndex.astype(jnp.int32)
    idxP = (idx.T.reshape(_HIST, _NBLK, _BB).transpose(1, 0, 2)
            .reshape(_NW, _ITERS, _BB))
    w2 = weights.reshape(_VOCAB // 2, 128)
    L = _embed(idxP, w2)
    return (
        L.reshape(_HIST, 8, _NBLK, 8, _BB)
        .transpose(2, 4, 0, 1, 3)
        .reshape(_BATCH, _HIST, _EMBED_DIM)
    )
